# baseline (speedup 1.0000x reference)
import numpy as np
import jax
import jax.numpy as jnp
from jax.experimental import pallas as pl
from jax.experimental.pallas import tpu as pltpu

_BB = 512


def _band_selector(n_in, n_out):
    sel = np.zeros((n_in * n_out, 5), np.float32)
    for i in range(n_in):
        for j in range(n_out):
            kw = i - j
            if 0 <= kw < 5:
                sel[i * n_out + j, kw] = 1.0
    return sel


_SEL1 = _band_selector(28, 24)
_SEL2 = _band_selector(12, 8)


def _pack_w1t(w1):
    r1 = w1.reshape(5, 5, 10).transpose(1, 0, 2).reshape(5, 50)
    t = jnp.dot(_SEL1, r1).reshape(28, 24, 5, 10).transpose(2, 0, 1, 3)
    t = t.reshape(5, 28, 12, 2, 10)
    even = jnp.pad(t[:, :, :, 0, :].reshape(5, 28, 120), ((0, 0),) * 2 + ((0, 8),))
    odd = jnp.pad(t[:, :, :, 1, :].reshape(5, 28, 120), ((0, 0),) * 2 + ((0, 8),))
    return jnp.concatenate([even, odd], axis=-1)


def _pack_w2t(w2):
    r2 = w2.reshape(5, 5, 10, 20).transpose(1, 0, 2, 3).reshape(5, 1000)
    t = jnp.dot(_SEL2, r2).reshape(12, 8, 5, 10, 20).transpose(2, 0, 3, 1, 4)
    t = t.reshape(5, 120, 4, 2, 20)
    pad = ((0, 0), (0, 8), (0, 48))
    even = jnp.pad(t[:, :, :, 0, :].reshape(5, 120, 80), pad)
    odd = jnp.pad(t[:, :, :, 1, :].reshape(5, 120, 80), pad)
    return jnp.concatenate([even, odd], axis=-1)


def _fused_kernel(x_ref, w1p_ref, b1_ref, w2p_ref, b2_ref,
                  wfc1_ref, bfc1_ref, wfc2_ref, bfc2_ref, out_ref):
    f32 = jnp.float32
    B = x_ref.shape[0]
    xt = jnp.transpose(x_ref[...], (1, 0, 2))
    xtp = jnp.pad(xt, ((0, 0), (0, 0), (0, 100)))
    w1p = w1p_ref[...]
    w2p = w2p_ref[...]
    wfc1 = wfc1_ref[...]

    lhs1 = jnp.concatenate([xtp[kh:kh + 24] for kh in range(5)],
                           axis=-1).reshape(24 * B, 640)
    acc = jnp.dot(lhs1, w1p, preferred_element_type=f32)
    acc = acc.reshape(24, B, 256)
    m = jnp.maximum(acc[:, :, :128], acc[:, :, 128:])
    m = m.reshape(12, 2, B, 128)
    m = jnp.maximum(m[:, 0], m[:, 1])
    h1 = jnp.maximum(m + b1_ref[...], 0.0)

    lhs2 = jnp.concatenate([h1[kh:kh + 8] for kh in range(5)],
                           axis=-1).reshape(8 * B, 640)
    acc2 = jnp.dot(lhs2, w2p, preferred_element_type=f32)
    acc2 = acc2.reshape(8, B, 256)
    m2 = jnp.maximum(acc2[:, :, :128], acc2[:, :, 128:])
    m2 = m2.reshape(4, 2, B, 128)
    m2 = jnp.maximum(m2[:, 0], m2[:, 1])
    h2 = jnp.maximum(m2 + b2_ref[...], 0.0)

    feats = jnp.concatenate([h2[0], h2[1], h2[2], h2[3]], axis=-1)
    z1 = bfc1_ref[...] + jnp.dot(feats, wfc1, preferred_element_type=f32)
    z1 = jnp.maximum(z1, 0.0)

    z2 = jnp.dot(z1, wfc2_ref[...], preferred_element_type=f32) + bfc2_ref[...]
    mz = jnp.max(z2, axis=-1, keepdims=True)
    e = jnp.exp(z2 - mz)
    out_ref[...] = (z2 - mz) - jnp.log(jnp.sum(e, axis=-1, keepdims=True))


def kernel(w1, b1, w2, b2, wfc1, bfc1, wfc2, bfc2, x):
    n = x.shape[0]
    nc = wfc2.shape[1]
    x = x.astype(jnp.float32).reshape(n, 28, 28)

    w1p = jnp.pad(_pack_w1t(w1), ((0, 0), (0, 100), (0, 0))).reshape(640, 256)
    w2p = _pack_w2t(w2).reshape(640, 256)
    b1l = jnp.pad(jnp.tile(b1, (1, 12)), ((0, 0), (0, 8)))
    b2l = jnp.pad(jnp.tile(b2, (1, 4)), ((0, 0), (0, 48)))
    wfc1p = jnp.pad(wfc1.reshape(4, 80, 50),
                    ((0, 0), (0, 48), (0, 0))).reshape(512, 50)

    n_pad = (-(-n // _BB)) * _BB
    if n_pad != n:
        x = jnp.pad(x, ((0, n_pad - n), (0, 0), (0, 0)))

    out = pl.pallas_call(
        _fused_kernel,
        out_shape=jax.ShapeDtypeStruct((n_pad, nc), jnp.float32),
        grid=(n_pad // _BB,),
        in_specs=[
            pl.BlockSpec((_BB, 28, 28), lambda i: (i, 0, 0)),
            pl.BlockSpec((640, 256), lambda i: (0, 0)),
            pl.BlockSpec((1, 128), lambda i: (0, 0)),
            pl.BlockSpec((640, 256), lambda i: (0, 0)),
            pl.BlockSpec((1, 128), lambda i: (0, 0)),
            pl.BlockSpec((512, 50), lambda i: (0, 0)),
            pl.BlockSpec((1, 50), lambda i: (0, 0)),
            pl.BlockSpec((50, nc), lambda i: (0, 0)),
            pl.BlockSpec((1, nc), lambda i: (0, 0)),
        ],
        out_specs=pl.BlockSpec((_BB, nc), lambda i: (i, 0)),
        compiler_params=pltpu.CompilerParams(
            dimension_semantics=("parallel",),
            vmem_limit_bytes=64 * 1024 * 1024,
        ),
    )(x, w1p, b1l, w2p, b2l, wfc1p, bfc1, wfc2, bfc2)

    return out[:n]


# speedup vs baseline: 1.0647x; 1.0647x over previous
import numpy as np
import jax
import jax.numpy as jnp
from jax.experimental import pallas as pl
from jax.experimental.pallas import tpu as pltpu

_BB = 512


def _band_selector(n_in, n_out):
    sel = np.zeros((n_in * n_out, 5), np.float32)
    for i in range(n_in):
        for j in range(n_out):
            kw = i - j
            if 0 <= kw < 5:
                sel[i * n_out + j, kw] = 1.0
    return sel


_SEL1 = _band_selector(28, 24)
_SEL2 = _band_selector(12, 8)


def _pack_w1t(w1):
    r1 = w1.reshape(5, 5, 10).transpose(1, 0, 2).reshape(5, 50)
    t = jnp.dot(_SEL1, r1).reshape(28, 24, 5, 10).transpose(2, 0, 1, 3)
    t = t.reshape(5, 28, 12, 2, 10)
    even = jnp.pad(t[:, :, :, 0, :].reshape(5, 28, 120), ((0, 0),) * 2 + ((0, 8),))
    odd = jnp.pad(t[:, :, :, 1, :].reshape(5, 28, 120), ((0, 0),) * 2 + ((0, 8),))
    return jnp.concatenate([even, odd], axis=-1)


def _pack_w2t(w2):
    r2 = w2.reshape(5, 5, 10, 20).transpose(1, 0, 2, 3).reshape(5, 1000)
    t = jnp.dot(_SEL2, r2).reshape(12, 8, 5, 10, 20).transpose(2, 0, 3, 1, 4)
    t = t.reshape(5, 120, 4, 2, 20)
    pad = ((0, 0), (0, 8), (0, 48))
    even = jnp.pad(t[:, :, :, 0, :].reshape(5, 120, 80), pad)
    odd = jnp.pad(t[:, :, :, 1, :].reshape(5, 120, 80), pad)
    return jnp.concatenate([even, odd], axis=-1)


def _fused_kernel(x_ref, w1p_ref, b1_ref, w2p_ref, b2_ref,
                  wfc1_ref, bfc1_ref, wfc2_ref, bfc2_ref, out_ref):
    f32 = jnp.float32
    B = x_ref.shape[0]
    xt = jnp.transpose(x_ref[...], (1, 0, 2))
    xtp = jnp.pad(xt, ((0, 0), (0, 0), (0, 100)))
    w1p = w1p_ref[...]
    w2p = w2p_ref[...]
    wfc1 = wfc1_ref[...]

    xcat = jnp.concatenate([xtp[0:27], xtp[1:28]], axis=-1)
    acc = jnp.dot(xcat[0:24].reshape(24 * B, 256), w1p[0],
                  preferred_element_type=f32)
    acc = acc + jnp.dot(xcat[2:26].reshape(24 * B, 256), w1p[1],
                        preferred_element_type=f32)
    acc = acc + jnp.dot(xtp[4:28].reshape(24 * B, 128), w1p[2, :128],
                        preferred_element_type=f32)
    acc = acc.reshape(24, B, 256)
    m = jnp.maximum(acc[:, :, :128], acc[:, :, 128:])
    m = m.reshape(12, 2, B, 128)
    m = jnp.maximum(m[:, 0], m[:, 1])
    h1 = jnp.maximum(m + b1_ref[...], 0.0)

    hcat = jnp.concatenate([h1[0:11], h1[1:12]], axis=-1)
    acc2 = jnp.dot(hcat[0:8].reshape(8 * B, 256), w2p[0],
                   preferred_element_type=f32)
    acc2 = acc2 + jnp.dot(hcat[2:10].reshape(8 * B, 256), w2p[1],
                          preferred_element_type=f32)
    acc2 = acc2 + jnp.dot(h1[4:12].reshape(8 * B, 128), w2p[2, :128],
                          preferred_element_type=f32)
    acc2 = acc2.reshape(8, B, 256)
    m2 = jnp.maximum(acc2[:, :, :128], acc2[:, :, 128:])
    m2 = m2.reshape(4, 2, B, 128)
    m2 = jnp.maximum(m2[:, 0], m2[:, 1])
    h2 = jnp.maximum(m2 + b2_ref[...], 0.0)

    f_a = jnp.concatenate([h2[0], h2[1]], axis=-1)
    f_b = jnp.concatenate([h2[2], h2[3]], axis=-1)
    z1 = bfc1_ref[...] + jnp.dot(f_a, wfc1[0], preferred_element_type=f32)
    z1 = z1 + jnp.dot(f_b, wfc1[1], preferred_element_type=f32)
    z1 = jnp.maximum(z1, 0.0)

    z2 = jnp.dot(z1, wfc2_ref[...], preferred_element_type=f32) + bfc2_ref[...]
    mz = jnp.max(z2, axis=-1, keepdims=True)
    e = jnp.exp(z2 - mz)
    out_ref[...] = (z2 - mz) - jnp.log(jnp.sum(e, axis=-1, keepdims=True))


def kernel(w1, b1, w2, b2, wfc1, bfc1, wfc2, bfc2, x):
    n = x.shape[0]
    nc = wfc2.shape[1]
    x = x.astype(jnp.float32).reshape(n, 28, 28)

    w1p = jnp.pad(_pack_w1t(w1),
                  ((0, 1), (0, 100), (0, 0))).reshape(3, 256, 256)
    w2p = jnp.pad(_pack_w2t(w2), ((0, 1), (0, 0), (0, 0))).reshape(3, 256, 256)
    b1l = jnp.pad(jnp.tile(b1, (1, 12)), ((0, 0), (0, 8)))
    b2l = jnp.pad(jnp.tile(b2, (1, 4)), ((0, 0), (0, 48)))
    wfc1p = jnp.pad(wfc1.reshape(4, 80, 50),
                    ((0, 0), (0, 48), (0, 0))).reshape(2, 256, 50)

    n_pad = (-(-n // _BB)) * _BB
    if n_pad != n:
        x = jnp.pad(x, ((0, n_pad - n), (0, 0), (0, 0)))

    out = pl.pallas_call(
        _fused_kernel,
        out_shape=jax.ShapeDtypeStruct((n_pad, nc), jnp.float32),
        grid=(n_pad // _BB,),
        in_specs=[
            pl.BlockSpec((_BB, 28, 28), lambda i: (i, 0, 0)),
            pl.BlockSpec((3, 256, 256), lambda i: (0, 0, 0)),
            pl.BlockSpec((1, 128), lambda i: (0, 0)),
            pl.BlockSpec((3, 256, 256), lambda i: (0, 0, 0)),
            pl.BlockSpec((1, 128), lambda i: (0, 0)),
            pl.BlockSpec((2, 256, 50), lambda i: (0, 0, 0)),
            pl.BlockSpec((1, 50), lambda i: (0, 0)),
            pl.BlockSpec((50, nc), lambda i: (0, 0)),
            pl.BlockSpec((1, nc), lambda i: (0, 0)),
        ],
        out_specs=pl.BlockSpec((_BB, nc), lambda i: (i, 0)),
        compiler_params=pltpu.CompilerParams(
            dimension_semantics=("parallel",),
            vmem_limit_bytes=64 * 1024 * 1024,
        ),
    )(x, w1p, b1l, w2p, b2l, wfc1p, bfc1, wfc2, bfc2)

    return out[:n]


# speedup vs baseline: 1.0648x; 1.0001x over previous
import numpy as np
import jax
import jax.numpy as jnp
from jax.experimental import pallas as pl
from jax.experimental.pallas import tpu as pltpu

_BB = 512


def _band_selector(n_in, n_out):
    sel = np.zeros((n_in * n_out, 5), np.float32)
    for i in range(n_in):
        for j in range(n_out):
            kw = i - j
            if 0 <= kw < 5:
                sel[i * n_out + j, kw] = 1.0
    return sel


_SEL1 = _band_selector(28, 24)
_SEL2 = _band_selector(12, 8)


def _pack_w1t(w1):
    r1 = w1.reshape(5, 5, 10).transpose(1, 0, 2).reshape(5, 50)
    t = jnp.dot(_SEL1, r1).reshape(28, 24, 5, 10).transpose(2, 0, 1, 3)
    t = t.reshape(5, 28, 12, 2, 10)
    even = jnp.pad(t[:, :, :, 0, :].reshape(5, 28, 120), ((0, 0),) * 2 + ((0, 8),))
    odd = jnp.pad(t[:, :, :, 1, :].reshape(5, 28, 120), ((0, 0),) * 2 + ((0, 8),))
    return jnp.concatenate([even, odd], axis=-1)


def _pack_w2t(w2):
    r2 = w2.reshape(5, 5, 10, 20).transpose(1, 0, 2, 3).reshape(5, 1000)
    t = jnp.dot(_SEL2, r2).reshape(12, 8, 5, 10, 20).transpose(2, 0, 3, 1, 4)
    t = t.reshape(5, 120, 4, 2, 20)
    pad = ((0, 0), (0, 8), (0, 48))
    even = jnp.pad(t[:, :, :, 0, :].reshape(5, 120, 80), pad)
    odd = jnp.pad(t[:, :, :, 1, :].reshape(5, 120, 80), pad)
    return jnp.concatenate([even, odd], axis=-1)


def _fused_kernel(x_ref, w1p_ref, b1_ref, w2p_ref, b2_ref,
                  wfc1_ref, bfc1_ref, wfc2_ref, bfc2_ref, out_ref):
    f32 = jnp.float32
    B = x_ref.shape[0]
    xt = jnp.transpose(x_ref[...], (1, 0, 2))
    xtp = jnp.pad(xt, ((0, 0), (0, 0), (0, 100)))
    w1p = w1p_ref[...]
    w2p = w2p_ref[...]
    wfc1 = wfc1_ref[...]

    xcat = jnp.concatenate([xtp[0:27], xtp[1:28]], axis=-1)
    acc = jnp.dot(xcat[0:24].reshape(24 * B, 256), w1p[0],
                  preferred_element_type=f32)
    acc = acc + jnp.dot(xcat[2:26].reshape(24 * B, 256), w1p[1],
                        preferred_element_type=f32)
    acc = acc + jnp.dot(xtp[4:28].reshape(24 * B, 128), w1p[2, :128],
                        preferred_element_type=f32)
    acc = acc.reshape(24, B, 256)
    m = jnp.maximum(acc[:, :, :128], acc[:, :, 128:])
    m = m.reshape(12, 2, B, 128)
    m = jnp.maximum(m[:, 0], m[:, 1])
    h1 = jnp.maximum(m + b1_ref[...], 0.0)

    hcat = jnp.concatenate([h1[0:11], h1[1:12]], axis=-1)
    acc2 = jnp.dot(hcat[0:8].reshape(8 * B, 256), w2p[0],
                   preferred_element_type=f32)
    acc2 = acc2 + jnp.dot(hcat[2:10].reshape(8 * B, 256), w2p[1],
                          preferred_element_type=f32)
    acc2 = acc2 + jnp.dot(h1[4:12].reshape(8 * B, 128), w2p[2, :128],
                          preferred_element_type=f32)
    acc2 = acc2.reshape(8, B, 256)
    m2 = jnp.maximum(acc2[:, :, :128], acc2[:, :, 128:])
    m2 = m2.reshape(4, 2, B, 128)
    m2 = jnp.maximum(m2[:, 0], m2[:, 1])
    h2 = jnp.maximum(m2 + b2_ref[...], 0.0)

    f_a = jnp.concatenate([h2[0], h2[1]], axis=-1)
    f_b = jnp.concatenate([h2[2], h2[3]], axis=-1)
    z1 = bfc1_ref[...] + jnp.dot(f_a, wfc1[0], preferred_element_type=f32)
    z1 = z1 + jnp.dot(f_b, wfc1[1], preferred_element_type=f32)
    z1 = jnp.maximum(z1, 0.0)

    z2 = jnp.dot(z1, wfc2_ref[...], preferred_element_type=f32) + bfc2_ref[...]
    mz = jnp.max(z2, axis=-1, keepdims=True)
    e = jnp.exp(z2 - mz)
    out_ref[...] = (z2 - mz) - jnp.log(jnp.sum(e, axis=-1, keepdims=True))


def kernel(w1, b1, w2, b2, wfc1, bfc1, wfc2, bfc2, x):
    n = x.shape[0]
    nc = wfc2.shape[1]
    x = x.astype(jnp.float32).reshape(n, 28, 28)

    w1p = jnp.pad(_pack_w1t(w1),
                  ((0, 1), (0, 100), (0, 0))).reshape(3, 256, 256)
    w2p = jnp.pad(_pack_w2t(w2), ((0, 1), (0, 0), (0, 0))).reshape(3, 256, 256)
    b1l = jnp.pad(jnp.tile(b1, (1, 12)), ((0, 0), (0, 8)))
    b2l = jnp.pad(jnp.tile(b2, (1, 4)), ((0, 0), (0, 48)))
    wfc1p = jnp.pad(wfc1.reshape(4, 80, 50),
                    ((0, 0), (0, 48), (0, 0))).reshape(2, 256, 50)

    n_pad = (-(-n // _BB)) * _BB
    if n_pad != n:
        x = jnp.pad(x, ((0, n_pad - n), (0, 0), (0, 0)))

    out = pl.pallas_call(
        _fused_kernel,
        out_shape=jax.ShapeDtypeStruct((n_pad, nc), jnp.float32),
        grid=(n_pad // _BB,),
        in_specs=[
            pl.BlockSpec((_BB, 28, 28), lambda i: (i, 0, 0)),
            pl.BlockSpec((3, 256, 256), lambda i: (0, 0, 0)),
            pl.BlockSpec((1, 128), lambda i: (0, 0)),
            pl.BlockSpec((3, 256, 256), lambda i: (0, 0, 0)),
            pl.BlockSpec((1, 128), lambda i: (0, 0)),
            pl.BlockSpec((2, 256, 50), lambda i: (0, 0, 0)),
            pl.BlockSpec((1, 50), lambda i: (0, 0)),
            pl.BlockSpec((50, nc), lambda i: (0, 0)),
            pl.BlockSpec((1, nc), lambda i: (0, 0)),
        ],
        out_specs=pl.BlockSpec((_BB, nc), lambda i: (i, 0)),
        compiler_params=pltpu.CompilerParams(
            dimension_semantics=("parallel",),
            vmem_limit_bytes=96 * 1024 * 1024,
        ),
    )(x, w1p, b1l, w2p, b2l, wfc1p, bfc1, wfc2, bfc2)

    return out[:n]
